# baseline (speedup 1.0000x reference)
import functools

import jax
import jax.numpy as jnp
from jax import lax
from jax.experimental import pallas as pl
from jax.experimental.pallas import tpu as pltpu
from jax.experimental.pallas import tpu_sc as plsc

_N = 100
_NP = 128
_E = 3200
_NW = 32
_NR = _NP // _NW
_VECS = _E // 16


def _sc_adj_kernel(src_hbm, dst_hbm, out_hbm, src_v, dst_v, acc_v):
    wid = lax.axis_index("s") * 2 + lax.axis_index("c")
    base = wid * _NR

    zeros = jnp.zeros((16,), jnp.float32)

    def zero_body(i, carry):
        acc_v[pl.ds(i * 16, 16)] = zeros
        return carry

    lax.fori_loop(0, _NR * _NP // 16, zero_body, 0)

    pltpu.sync_copy(src_hbm, src_v)
    pltpu.sync_copy(dst_hbm, dst_v)

    ones = jnp.full((16,), 1.0, jnp.float32)

    def edge_body(i, carry):
        s = src_v[pl.ds(i * 16, 16)]
        d = dst_v[pl.ds(i * 16, 16)]
        rel = d - base
        m = (rel >= 0) & (rel < _NR)
        idx = jnp.where(m, rel * _NP + s, 0)
        plsc.addupdate_scatter(acc_v, [idx], ones, mask=m)
        return carry

    lax.fori_loop(0, _VECS, edge_body, 0)

    pltpu.sync_copy(acc_v, out_hbm.at[pl.ds(base * _NP, _NR * _NP)])


def _sc_build_adj(src, dst):
    mesh = plsc.VectorSubcoreMesh(core_axis_name="c", subcore_axis_name="s")
    return pl.kernel(
        _sc_adj_kernel,
        out_type=jax.ShapeDtypeStruct((_NP * _NP,), jnp.float32),
        mesh=mesh,
        compiler_params=pltpu.CompilerParams(needs_layout_passes=False),
        scratch_types=[
            pltpu.VMEM((_E,), jnp.int32),
            pltpu.VMEM((_E,), jnp.int32),
            pltpu.VMEM((_NR * _NP,), jnp.float32),
        ],
    )(src, dst)


def _gcn_dense_kernel(adj_ref, x_ref, w1_ref, b1_ref, w2_ref, b2_ref,
                      out_ref):
    f32 = jnp.float32
    hi = lax.Precision.HIGHEST

    adj = adj_ref[:]
    eye = (lax.broadcasted_iota(jnp.int32, (_NP, _NP), 0)
           == lax.broadcasted_iota(jnp.int32, (_NP, _NP), 1)).astype(f32)

    deg = jnp.sum(adj, axis=1, keepdims=True) + 1.0
    dinv = lax.rsqrt(deg)
    dmat = eye * dinv
    a = jnp.dot(jnp.dot(dmat, adj + eye, precision=hi), dmat, precision=hi)

    xw = jnp.dot(x_ref[:], w1_ref[:], precision=hi)
    h = jnp.maximum(jnp.dot(a, xw, precision=hi) + b1_ref[:], 0.0)

    ah = jnp.dot(a, h, precision=hi)
    out_ref[:] = jnp.dot(ah, w2_ref[:], precision=hi) + b2_ref[:]


@jax.jit
def kernel(x, edge_index, W1, b1, W2, b2):
    src = edge_index[0].astype(jnp.int32)
    dst = edge_index[1].astype(jnp.int32)
    x_pad = jnp.zeros((_NP, x.shape[1]), jnp.float32).at[:_N].set(x)

    adj = _sc_build_adj(src, dst).reshape(_NP, _NP)

    out = pl.pallas_call(
        _gcn_dense_kernel,
        out_shape=jax.ShapeDtypeStruct((_NP, W2.shape[1]), jnp.float32),
    )(adj, x_pad, W1, b1.reshape(1, -1), W2, b2.reshape(1, -1))
    return out[:_N].reshape(_N * W2.shape[1])


# speedup vs baseline: 4.2196x; 4.2196x over previous
import jax
import jax.numpy as jnp
from jax import lax
from jax.experimental import pallas as pl

_N = 100
_NP = 128
_E = 3200


def _gcn_tc_kernel(edge_ref, x_ref, w1_ref, b1_ref, w2_ref, b2_ref, out_ref):
    f32 = jnp.float32
    hi = lax.Precision.HIGHEST

    node_iota = lax.broadcasted_iota(jnp.int32, (_NP, _E), 0)
    src_row = edge_ref[0:1, :]
    dst_row = edge_ref[1:2, :]
    Dt = (dst_row == node_iota).astype(f32)
    St = (src_row == node_iota).astype(f32)

    adj = lax.dot_general(Dt, St, (((1,), (1,)), ((), ())),
                          preferred_element_type=f32)

    eye = (lax.broadcasted_iota(jnp.int32, (_NP, _NP), 0)
           == lax.broadcasted_iota(jnp.int32, (_NP, _NP), 1)).astype(f32)
    deg = jnp.sum(Dt, axis=1, keepdims=True) + 1.0
    dinv = lax.rsqrt(deg)
    dmat = eye * dinv
    a = jnp.dot(jnp.dot(dmat, adj + eye, precision=hi), dmat, precision=hi)
    a_ss = a[:_N, :_N]

    xw = jnp.dot(x_ref[:], w1_ref[:], precision=hi)
    h = jnp.maximum(jnp.dot(a_ss, xw, precision=hi) + b1_ref[:].reshape(1, -1),
                    0.0)

    ah = jnp.dot(a_ss, h, precision=hi)
    out_ref[:] = jnp.dot(ah, w2_ref[:], precision=hi) + b2_ref[:].reshape(1, -1)


@jax.jit
def kernel(x, edge_index, W1, b1, W2, b2):
    out = pl.pallas_call(
        _gcn_tc_kernel,
        out_shape=jax.ShapeDtypeStruct((_N, W2.shape[1]), jnp.float32),
    )(edge_index.astype(jnp.int32), x, W1, b1, W2, b2)
    return out.reshape(_N * W2.shape[1])
